# baseline (speedup 1.0000x reference)
import functools

import jax
import jax.numpy as jnp
import numpy as np
from jax import lax
from jax.experimental import pallas as pl
from jax.experimental.pallas import tpu as pltpu
from jax.experimental.pallas import tpu_sc as plsc

N = 1_000_000
K = N // 10
WORKERS = 32
E = 32_768
N_PAD = WORKERS * E
VREGS = E // 16
B = 256
UNROLL = 8

_F32_MAX_U = np.uint32(0x7FFFFFFF)
_TOPBIT = np.uint32(0x80000000)


def _keyify(v):
  b = plsc.bitcast(v, jnp.uint32)
  neg = b >= _TOPBIT
  t = jnp.where(neg, jnp.uint32(0), _F32_MAX_U)
  return jnp.bitwise_xor(b, t)


def _wid():
  return lax.axis_index("c") * 16 + lax.axis_index("s")


NSUB = 4


def _hist_body(r, in_hbm, pref_hbm, out_hbm, data_v, hist_v, dump_v, pref_v):
  w = _wid()
  lane_i = lax.iota(jnp.int32, 16)
  zeros_i = jnp.zeros((16,), jnp.int32)
  ones_i = jnp.ones((16,), jnp.int32)
  shift = jnp.uint32(24 - 8 * r)

  pltpu.sync_copy(in_hbm.at[pl.ds(w * E, E)], data_v)
  if r > 0:
    pltpu.sync_copy(pref_hbm, pref_v)
    pref_b = plsc.bitcast(pref_v[...], jnp.uint32)

  def clr(i, _):
    hist_v[pl.ds(i * 16, 16)] = zeros_i
    return 0
  lax.fori_loop(0, NSUB * B, clr, 0)

  def scan_body(i, _):
    for j in range(UNROLL):
      v = data_v[pl.ds((i * UNROLL + j) * 16, 16)]
      u = _keyify(v)
      d = plsc.bitcast(
          jnp.bitwise_and(jnp.right_shift(u, shift), jnp.uint32(0xFF)),
          jnp.int32)
      idx = jnp.left_shift(d, 4) + lane_i + jnp.int32((j % NSUB) * B * 16)
      if r == 0:
        plsc.addupdate_scatter(hist_v, [idx], ones_i)
      else:
        m = jnp.right_shift(u, jnp.uint32(32 - 8 * r)) == pref_b
        plsc.addupdate_scatter(hist_v, [idx], ones_i, mask=m)
    return 0
  lax.fori_loop(0, VREGS // UNROLL, scan_body, 0)

  def mrg(i, _):
    dump_v[pl.ds(i * 16, 16)] = (
        hist_v[pl.ds(i * 16, 16)] + hist_v[pl.ds(B * 16 + i * 16, 16)]
    ) + (
        hist_v[pl.ds(2 * B * 16 + i * 16, 16)]
        + hist_v[pl.ds(3 * B * 16 + i * 16, 16)]
    )
    return 0
  lax.fori_loop(0, B, mrg, 0)

  pltpu.sync_copy(dump_v, out_hbm.at[w])


def _sum_body(in_hbm, thr_hbm, out_hbm, data_v, thr_v, stagef_v):
  w = _wid()
  zeros_f = jnp.zeros((16,), jnp.float32)

  pltpu.sync_copy(in_hbm.at[pl.ds(w * E, E)], data_v)
  pltpu.sync_copy(thr_hbm, thr_v)
  ustar_b = plsc.bitcast(thr_v[...], jnp.uint32)

  def sum_body(i, acc):
    for j in range(UNROLL):
      v = data_v[pl.ds((i * UNROLL + j) * 16, 16)]
      u = _keyify(v)
      acc = acc + jnp.where(u > ustar_b, v, zeros_f)
    return acc
  accf = lax.fori_loop(0, VREGS // UNROLL, sum_body, zeros_f)
  stagef_v[...] = accf
  pltpu.sync_copy(stagef_v, out_hbm.at[w])


def _make_hist_kernel(r):
  mesh = plsc.VectorSubcoreMesh(core_axis_name="c", subcore_axis_name="s")
  return pl.kernel(
      functools.partial(_hist_body, r),
      out_type=jax.ShapeDtypeStruct((WORKERS, B * 16), jnp.int32),
      mesh=mesh,
      scratch_types=[
          pltpu.VMEM((E,), jnp.float32),
          pltpu.VMEM((NSUB * B * 16,), jnp.int32),
          pltpu.VMEM((B * 16,), jnp.int32),
          pltpu.VMEM((16,), jnp.int32),
      ],
      compiler_params=pltpu.CompilerParams(needs_layout_passes=False),
  )


def _make_sum_kernel():
  mesh = plsc.VectorSubcoreMesh(core_axis_name="c", subcore_axis_name="s")
  return pl.kernel(
      _sum_body,
      out_type=jax.ShapeDtypeStruct((WORKERS, 16), jnp.float32),
      mesh=mesh,
      scratch_types=[
          pltpu.VMEM((E,), jnp.float32),
          pltpu.VMEM((16,), jnp.int32),
          pltpu.VMEM((16,), jnp.float32),
      ],
      compiler_params=pltpu.CompilerParams(needs_layout_passes=False),
  )


@jax.jit
def _es_pallas(xp):
  bins = jnp.arange(B, dtype=jnp.int32)
  k_rem = jnp.int32(K)
  prefix = jnp.uint32(0)
  zeros16 = jnp.zeros((16,), jnp.int32)

  for r in range(4):
    pref_vec = jnp.broadcast_to(
        lax.bitcast_convert_type(prefix, jnp.int32), (16,))
    hists = _make_hist_kernel(r)(xp, pref_vec if r > 0 else zeros16)
    merged = jnp.sum(hists.reshape(WORKERS, B, 16), axis=(0, 2),
                     dtype=jnp.int32)
    suf = jnp.cumsum(merged[::-1])[::-1]
    dstar = jnp.max(jnp.where(suf >= k_rem, bins, 0))
    above = jnp.where(
        dstar < B - 1,
        suf[jnp.minimum(dstar + 1, B - 1)],
        0,
    )
    k_rem = k_rem - above
    prefix = jnp.bitwise_or(
        jnp.left_shift(prefix, jnp.uint32(8)), dstar.astype(jnp.uint32))

  thr_vec = jnp.broadcast_to(
      lax.bitcast_convert_type(prefix, jnp.int32), (16,))
  parts = _make_sum_kernel()(xp, thr_vec)
  total = jnp.sum(parts, dtype=jnp.float32)

  bb = jnp.where(prefix >= _TOPBIT, prefix,
                 jnp.bitwise_xor(prefix, _F32_MAX_U))
  vstar = lax.bitcast_convert_type(bb, jnp.float32)
  ties = k_rem.astype(jnp.float32)
  return -(total + ties * vstar) * np.float32(1.0 / K)


def kernel(input):
  pad = jnp.full((N_PAD - N,), jnp.inf, dtype=jnp.float32)
  xp = jnp.concatenate([input, pad])
  return _es_pallas(xp)


# speedup vs baseline: 1.1570x; 1.1570x over previous
import functools

import jax
import jax.numpy as jnp
import numpy as np
from jax import lax
from jax.experimental import pallas as pl
from jax.experimental.pallas import tpu as pltpu
from jax.experimental.pallas import tpu_sc as plsc

N = 1_000_000
K = N // 10
WORKERS = 32
E = 32_768
N_PAD = WORKERS * E
VREGS = E // 16
B = 65_536
UNROLL = 8

_F32_MAX_U = np.uint32(0x7FFFFFFF)
_TOPBIT = np.uint32(0x80000000)


def _keyify(v):
  b = plsc.bitcast(v, jnp.uint32)
  neg = b >= _TOPBIT
  t = jnp.where(neg, jnp.uint32(0), _F32_MAX_U)
  return jnp.bitwise_xor(b, t)


def _wid():
  return lax.axis_index("c") * 16 + lax.axis_index("s")


def _hist_body(r, in_hbm, pref_hbm, out_hbm, data_v, hist_v, pref_v):
  w = _wid()
  zeros_i = jnp.zeros((16,), jnp.int32)
  ones_i = jnp.ones((16,), jnp.int32)

  pltpu.sync_copy(in_hbm.at[pl.ds(w * E, E)], data_v)
  if r > 0:
    pltpu.sync_copy(pref_hbm, pref_v)
    pref_b = plsc.bitcast(pref_v[...], jnp.uint32)

  def clr(i, _):
    hist_v[pl.ds(i * 16, 16)] = zeros_i
    return 0
  lax.fori_loop(0, B // 16, clr, 0)

  def scan_body(i, _):
    for j in range(UNROLL):
      v = data_v[pl.ds((i * UNROLL + j) * 16, 16)]
      u = _keyify(v)
      if r == 0:
        d = plsc.bitcast(jnp.right_shift(u, jnp.uint32(16)), jnp.int32)
        plsc.addupdate_scatter(hist_v, [d], ones_i)
      else:
        d = plsc.bitcast(
            jnp.bitwise_and(u, jnp.uint32(0xFFFF)), jnp.int32)
        m = jnp.right_shift(u, jnp.uint32(16)) == pref_b
        plsc.addupdate_scatter(hist_v, [d], ones_i, mask=m)
    return 0
  lax.fori_loop(0, VREGS // UNROLL, scan_body, 0)

  pltpu.sync_copy(hist_v, out_hbm.at[w])


def _sum_body(in_hbm, thr_hbm, out_hbm, data_v, thr_v, stagef_v):
  w = _wid()
  zeros_f = jnp.zeros((16,), jnp.float32)

  pltpu.sync_copy(in_hbm.at[pl.ds(w * E, E)], data_v)
  pltpu.sync_copy(thr_hbm, thr_v)
  ustar_b = plsc.bitcast(thr_v[...], jnp.uint32)

  def sum_body(i, acc):
    for j in range(UNROLL):
      v = data_v[pl.ds((i * UNROLL + j) * 16, 16)]
      u = _keyify(v)
      acc = acc + jnp.where(u > ustar_b, v, zeros_f)
    return acc
  accf = lax.fori_loop(0, VREGS // UNROLL, sum_body, zeros_f)
  stagef_v[...] = accf
  pltpu.sync_copy(stagef_v, out_hbm.at[w])


def _make_hist_kernel(r):
  mesh = plsc.VectorSubcoreMesh(core_axis_name="c", subcore_axis_name="s")
  return pl.kernel(
      functools.partial(_hist_body, r),
      out_type=jax.ShapeDtypeStruct((WORKERS, B), jnp.int32),
      mesh=mesh,
      scratch_types=[
          pltpu.VMEM((E,), jnp.float32),
          pltpu.VMEM((B,), jnp.int32),
          pltpu.VMEM((16,), jnp.int32),
      ],
      compiler_params=pltpu.CompilerParams(needs_layout_passes=False),
  )


def _make_sum_kernel():
  mesh = plsc.VectorSubcoreMesh(core_axis_name="c", subcore_axis_name="s")
  return pl.kernel(
      _sum_body,
      out_type=jax.ShapeDtypeStruct((WORKERS, 16), jnp.float32),
      mesh=mesh,
      scratch_types=[
          pltpu.VMEM((E,), jnp.float32),
          pltpu.VMEM((16,), jnp.int32),
          pltpu.VMEM((16,), jnp.float32),
      ],
      compiler_params=pltpu.CompilerParams(needs_layout_passes=False),
  )


@jax.jit
def _es_pallas(xp):
  bins = jnp.arange(B, dtype=jnp.int32)
  k_rem = jnp.int32(K)
  prefix = jnp.uint32(0)
  zeros16 = jnp.zeros((16,), jnp.int32)

  for r in range(2):
    pref_vec = jnp.broadcast_to(
        lax.bitcast_convert_type(prefix, jnp.int32), (16,))
    hists = _make_hist_kernel(r)(xp, pref_vec if r > 0 else zeros16)
    merged = jnp.sum(hists, axis=0, dtype=jnp.int32)
    suf = jnp.cumsum(merged[::-1])[::-1]
    dstar = jnp.max(jnp.where(suf >= k_rem, bins, 0))
    above = jnp.where(
        dstar < B - 1,
        suf[jnp.minimum(dstar + 1, B - 1)],
        0,
    )
    k_rem = k_rem - above
    prefix = jnp.bitwise_or(
        jnp.left_shift(prefix, jnp.uint32(16)), dstar.astype(jnp.uint32))

  thr_vec = jnp.broadcast_to(
      lax.bitcast_convert_type(prefix, jnp.int32), (16,))
  parts = _make_sum_kernel()(xp, thr_vec)
  total = jnp.sum(parts, dtype=jnp.float32)

  bb = jnp.where(prefix >= _TOPBIT, prefix,
                 jnp.bitwise_xor(prefix, _F32_MAX_U))
  vstar = lax.bitcast_convert_type(bb, jnp.float32)
  ties = k_rem.astype(jnp.float32)
  return -(total + ties * vstar) * np.float32(1.0 / K)


def kernel(input):
  pad = jnp.full((N_PAD - N,), jnp.inf, dtype=jnp.float32)
  xp = jnp.concatenate([input, pad])
  return _es_pallas(xp)
